# baseline (speedup 1.0000x reference)
import jax
import jax.numpy as jnp
from jax.experimental import pallas as pl
from jax.experimental.pallas import tpu as pltpu
from jax.experimental.pallas import tpu_sc as plsc

_LANES = 16


def kernel(x, pos_table):
    B, S, D = x.shape
    BR = 8

    mesh = plsc.VectorSubcoreMesh(core_axis_name="c", subcore_axis_name="s")

    @pl.kernel(out_type=jax.ShapeDtypeStruct((B, S, D), x.dtype), mesh=mesh)
    def sc_add(x_hbm, p_hbm, o_hbm):
        def body(x_vmem, p_vmem, o_vmem):
            @pl.loop(0, BR)
            def _(r):
                @pl.loop(0, D, step=_LANES)
                def _(c):
                    o_vmem[0, r, pl.ds(c, _LANES)] = (
                        x_vmem[0, r, pl.ds(c, _LANES)]
                        + p_vmem[r, pl.ds(c, _LANES)]
                    )

        pltpu.emit_pipeline(
            body,
            grid=(B, S // BR),
            in_specs=[
                pl.BlockSpec((1, BR, D), index_map=lambda b, i: (b, i, 0)),
                pl.BlockSpec((BR, D), index_map=lambda b, i: (i, 0)),
            ],
            out_specs=[pl.BlockSpec((1, BR, D), index_map=lambda b, i: (b, i, 0))],
            core_axis_name=("c", "s"),
            dimension_semantics=(pltpu.PARALLEL, pltpu.PARALLEL),
        )(x_hbm, p_hbm, o_hbm)

    return sc_add(x, pos_table)
